# baseline (speedup 1.0000x reference)
import functools

import numpy as np
import jax
from jax import lax
import jax.numpy as jnp
from jax.experimental import pallas as pl
from jax.experimental.pallas import tpu as pltpu
from jax.experimental.pallas import tpu_sc as plsc

_K = 8192
_FEAT = 256
_NCLS = 65
_B = 256
_DIN = 2048
_M = 0.999
_T = 0.07

_TJ = 512
_NJ = _K // _TJ

_PERM = np.array([
    183, 138, 166, 19, 76, 158, 219, 118, 143, 54, 237, 189, 227, 149, 90, 30,
    7, 96, 139, 155, 131, 121, 115, 6, 35, 23, 58, 228, 128, 230, 16, 21,
    194, 213, 156, 220, 77, 154, 160, 94, 116, 61, 229, 38, 3, 185, 105, 132,
    81, 26, 32, 64, 37, 56, 51, 2, 193, 122, 248, 63, 133, 52, 20, 89,
    202, 95, 44, 47, 123, 239, 79, 84, 222, 144, 157, 135, 50, 242, 140, 78,
    179, 72, 163, 191, 83, 42, 62, 254, 152, 69, 235, 53, 247, 234, 245, 223,
    148, 172, 215, 0, 201, 226, 145, 8, 208, 203, 167, 169, 159, 251, 109, 181,
    22, 178, 13, 29, 99, 110, 244, 34, 70, 175, 18, 103, 196, 141, 252, 232,
    86, 142, 75, 233, 198, 187, 206, 91, 243, 111, 24, 113, 1, 65, 48, 5,
    238, 45, 199, 165, 150, 49, 173, 214, 236, 33, 216, 74, 55, 182, 136, 60,
    204, 119, 57, 124, 27, 112, 129, 249, 209, 151, 10, 134, 192, 246, 186, 93,
    176, 161, 68, 146, 240, 15, 217, 73, 241, 40, 210, 67, 88, 102, 107, 66,
    80, 100, 120, 211, 147, 71, 207, 17, 59, 184, 98, 225, 108, 114, 36, 125,
    101, 218, 180, 92, 171, 153, 28, 46, 9, 104, 200, 255, 117, 221, 4, 177,
    170, 190, 130, 12, 168, 195, 188, 87, 85, 212, 14, 174, 82, 31, 106, 127,
    250, 162, 126, 164, 231, 97, 224, 41, 253, 137, 197, 25, 43, 39, 11, 205,
], dtype=np.int32)

_PERM_ONEHOT = np.zeros((_B, _B), dtype=np.float32)
_PERM_ONEHOT[np.arange(_B), _PERM] = 1.0


def _k1_body(im_q_ref, im_k_ref, W_q_ref, b_q_ref, W_cls_ref, b_cls_ref,
             W_k_ref, b_k_ref, W_cls_k_ref, b_cls_k_ref, P_ref,
             feats_q_ref, logits_q_ref, q_ref, k_ref, kT_ref,
             logits_k_ref, l_pos_ref, probs_ref, pseudo_ref):
    im_q = im_q_ref[...]
    W_q = W_q_ref[...]
    b_q = b_q_ref[...]
    feats_q = jnp.dot(im_q, W_q, preferred_element_type=jnp.float32) + b_q
    feats_q_ref[...] = feats_q
    logits_q_ref[...] = (
        jnp.dot(feats_q, W_cls_ref[...], preferred_element_type=jnp.float32)
        + b_cls_ref[...])
    nq = jnp.sqrt(jnp.sum(feats_q * feats_q, axis=1, keepdims=True))
    q = feats_q / jnp.maximum(nq, 1e-12)
    q_ref[...] = q

    W_k2 = W_k_ref[...] * _M + W_q * (1.0 - _M)
    b_k2 = b_k_ref[...] * _M + b_q * (1.0 - _M)
    W_cls_k2 = W_cls_k_ref[...] * _M + W_cls_ref[...] * (1.0 - _M)
    b_cls_k2 = b_cls_k_ref[...] * _M + b_cls_ref[...] * (1.0 - _M)

    feats_k = jnp.dot(im_k_ref[...], W_k2, preferred_element_type=jnp.float32) + b_k2
    logits_k_u = (jnp.dot(feats_k, W_cls_k2, preferred_element_type=jnp.float32)
                  + b_cls_k2)
    logits_k = jnp.dot(P_ref[...], logits_k_u,
                       preferred_element_type=jnp.float32)
    logits_k_ref[...] = logits_k
    nk = jnp.sqrt(jnp.sum(feats_k * feats_k, axis=1, keepdims=True))
    k = feats_k / jnp.maximum(nk, 1e-12)
    k_ref[...] = k
    kT_ref[...] = k.T

    l_pos_ref[...] = jnp.sum(q * k, axis=1, keepdims=True) * (1.0 / _T)

    m = jnp.max(logits_k_u, axis=1, keepdims=True)
    e = jnp.exp(logits_k_u - m)
    p_u = e / jnp.sum(e, axis=1, keepdims=True)
    probs_ref[...] = jnp.dot(P_ref[...], p_u,
                             preferred_element_type=jnp.float32)
    col = jax.lax.broadcasted_iota(jnp.int32, (_B, _NCLS), 1)
    pseudo_ref[...] = jnp.min(jnp.where(logits_k_u == m, col, _NCLS),
                              axis=1, keepdims=True)


def _k2_body(memf_ref, q_ref, l_pos_ref, lnn_ref, lins_ref, carry_ref):
    j = pl.program_id(0)

    @pl.when(j == 0)
    def _():
        carry_ref[...] = l_pos_ref[...]

    @pl.when(j < _NJ)
    def _():
        a = memf_ref[...]
        b = memf_ref[:, pl.ds(j * _TJ, _TJ)]
        lnn_ref[...] = jax.lax.dot_general(
            a, b, (((0,), (0,)), ((), ())), preferred_element_type=jnp.float32)
        lneg = (jnp.dot(q_ref[...], b, preferred_element_type=jnp.float32)
                * (1.0 / _T))
        lins_ref[...] = jnp.concatenate(
            [carry_ref[...], lneg[:, :_TJ - 1]], axis=1)
        carry_ref[...] = lneg[:, _TJ - 1:]

    @pl.when(j == _NJ)
    def _():
        lins_ref[:, 0:1] = carry_ref[...]


_NC, _NS = 2, 16
_NW = _NC * _NS
_ROWS_W = _FEAT // _NW
_SLOTS_W = _K // _NW


def _sc_bank_body(memf, kT, memp, probs, meml, pseudo, memi, idxs,
                  memf_out, memp_out, meml_out, memi_out):
    wid = lax.axis_index("s") * _NC + lax.axis_index("c")
    base = wid * _SLOTS_W

    @pl.when(wid == 0)
    def _():
        pltpu.sync_copy(pseudo, meml_out.at[pl.ds(0, _B)])
        pltpu.sync_copy(idxs, memi_out.at[pl.ds(0, _B)])
        pltpu.sync_copy(probs, memp_out.at[pl.ds(0, _B), :])

    @pl.when(wid != 0)
    def _():
        pltpu.sync_copy(meml.at[pl.ds(base, _SLOTS_W)],
                        meml_out.at[pl.ds(base, _SLOTS_W)])
        pltpu.sync_copy(memi.at[pl.ds(base, _SLOTS_W)],
                        memi_out.at[pl.ds(base, _SLOTS_W)])
        pltpu.sync_copy(memp.at[pl.ds(base, _SLOTS_W), :],
                        memp_out.at[pl.ds(base, _SLOTS_W), :])

    r0 = wid * _ROWS_W
    pltpu.sync_copy(kT.at[pl.ds(r0, _ROWS_W), :],
                    memf_out.at[pl.ds(r0, _ROWS_W), pl.ds(0, _B)])
    pltpu.sync_copy(memf.at[pl.ds(r0, _ROWS_W), pl.ds(_B, _K - _B)],
                    memf_out.at[pl.ds(r0, _ROWS_W), pl.ds(_B, _K - _B)])


def _sc_bank_update(mem_feat, kT, mem_probs, probs, mem_labels, pseudo,
                    mem_index, idxs_i32):
    f32 = jnp.float32
    mesh = plsc.VectorSubcoreMesh(core_axis_name="c", subcore_axis_name="s")
    fn = pl.kernel(
        _sc_bank_body,
        mesh=mesh,
        out_type=(
            jax.ShapeDtypeStruct((_FEAT, _K), f32),
            jax.ShapeDtypeStruct((_K, _NCLS), f32),
            jax.ShapeDtypeStruct((_K,), mem_labels.dtype),
            jax.ShapeDtypeStruct((_K,), mem_index.dtype),
        ),
    )
    return fn(mem_feat, kT, mem_probs, probs, mem_labels, pseudo, mem_index,
              idxs_i32)


def kernel(im_q, im_k, idxs, W_q, b_q, W_cls, b_cls, W_k, b_k, W_cls_k,
           b_cls_k, mem_feat, mem_labels, mem_probs, mem_index):
    f32 = jnp.float32

    P = jnp.asarray(_PERM_ONEHOT)
    (feats_q, logits_q, q, k, kT, logits_k, l_pos, probs,
     pseudo_u) = pl.pallas_call(
        _k1_body,
        out_shape=(
            jax.ShapeDtypeStruct((_B, _FEAT), f32),
            jax.ShapeDtypeStruct((_B, _NCLS), f32),
            jax.ShapeDtypeStruct((_B, _FEAT), f32),
            jax.ShapeDtypeStruct((_B, _FEAT), f32),
            jax.ShapeDtypeStruct((_FEAT, _B), f32),
            jax.ShapeDtypeStruct((_B, _NCLS), f32),
            jax.ShapeDtypeStruct((_B, 1), f32),
            jax.ShapeDtypeStruct((_B, _NCLS), f32),
            jax.ShapeDtypeStruct((_B, 1), jnp.int32),
        ),
    )(im_q, im_k, W_q, b_q.reshape(1, _FEAT), W_cls, b_cls.reshape(1, _NCLS),
      W_k, b_k.reshape(1, _FEAT), W_cls_k, b_cls_k.reshape(1, _NCLS), P)

    pseudo = pseudo_u[_PERM].reshape(_B)

    l_neg_near, logits_ins = pl.pallas_call(
        _k2_body,
        grid=(_NJ + 1,),
        scratch_shapes=[pltpu.VMEM((_B, 1), jnp.float32)],
        in_specs=[
            pl.BlockSpec((_FEAT, _K), lambda j: (0, 0)),
            pl.BlockSpec((_B, _FEAT), lambda j: (0, 0)),
            pl.BlockSpec((_B, 1), lambda j: (0, 0)),
        ],
        out_specs=[
            pl.BlockSpec((_K, _TJ), lambda j: (0, jnp.minimum(j, _NJ - 1))),
            pl.BlockSpec((_B, _TJ), lambda j: (0, j)),
        ],
        out_shape=(
            jax.ShapeDtypeStruct((_K, _K), f32),
            jax.ShapeDtypeStruct((_B, _K + 1), f32),
        ),
    )(mem_feat, q, l_pos)

    mem_feat_new, mem_probs_new, meml_new, memi_new = _sc_bank_update(
        mem_feat, kT, mem_probs, probs, mem_labels, pseudo, mem_index,
        idxs.astype(mem_index.dtype))

    return (feats_q, logits_q, logits_ins, k, logits_k, l_neg_near,
            mem_feat_new, meml_new, mem_probs_new, memi_new)


# speedup vs baseline: 3.2470x; 3.2470x over previous
import numpy as np
import jax
import jax.numpy as jnp
from jax.experimental import pallas as pl
from jax.experimental.pallas import tpu as pltpu

_K = 8192
_FEAT = 256
_NCLS = 65
_B = 256
_DIN = 2048
_M = 0.999
_T = 0.07

_TJ = 256
_NJ = _K // _TJ

_PERM = np.array([
    183, 138, 166, 19, 76, 158, 219, 118, 143, 54, 237, 189, 227, 149, 90, 30,
    7, 96, 139, 155, 131, 121, 115, 6, 35, 23, 58, 228, 128, 230, 16, 21,
    194, 213, 156, 220, 77, 154, 160, 94, 116, 61, 229, 38, 3, 185, 105, 132,
    81, 26, 32, 64, 37, 56, 51, 2, 193, 122, 248, 63, 133, 52, 20, 89,
    202, 95, 44, 47, 123, 239, 79, 84, 222, 144, 157, 135, 50, 242, 140, 78,
    179, 72, 163, 191, 83, 42, 62, 254, 152, 69, 235, 53, 247, 234, 245, 223,
    148, 172, 215, 0, 201, 226, 145, 8, 208, 203, 167, 169, 159, 251, 109, 181,
    22, 178, 13, 29, 99, 110, 244, 34, 70, 175, 18, 103, 196, 141, 252, 232,
    86, 142, 75, 233, 198, 187, 206, 91, 243, 111, 24, 113, 1, 65, 48, 5,
    238, 45, 199, 165, 150, 49, 173, 214, 236, 33, 216, 74, 55, 182, 136, 60,
    204, 119, 57, 124, 27, 112, 129, 249, 209, 151, 10, 134, 192, 246, 186, 93,
    176, 161, 68, 146, 240, 15, 217, 73, 241, 40, 210, 67, 88, 102, 107, 66,
    80, 100, 120, 211, 147, 71, 207, 17, 59, 184, 98, 225, 108, 114, 36, 125,
    101, 218, 180, 92, 171, 153, 28, 46, 9, 104, 200, 255, 117, 221, 4, 177,
    170, 190, 130, 12, 168, 195, 188, 87, 85, 212, 14, 174, 82, 31, 106, 127,
    250, 162, 126, 164, 231, 97, 224, 41, 253, 137, 197, 25, 43, 39, 11, 205,
], dtype=np.int32)

_PERM_ONEHOT = np.zeros((_B, _B), dtype=np.float32)
_PERM_ONEHOT[np.arange(_B), _PERM] = 1.0


def _k1_body(im_q_ref, im_k_ref, W_q_ref, b_q_ref, W_cls_ref, b_cls_ref,
             W_k_ref, b_k_ref, W_cls_k_ref, b_cls_k_ref, P_ref,
             feats_q_ref, logits_q_ref, q_ref, k_ref, kT_ref,
             logits_k_ref, l_pos_ref, probs_ref, pseudo_ref):
    im_q = im_q_ref[...]
    W_q = W_q_ref[...]
    b_q = b_q_ref[...]
    feats_q = jnp.dot(im_q, W_q, preferred_element_type=jnp.float32) + b_q
    feats_q_ref[...] = feats_q
    logits_q_ref[...] = (
        jnp.dot(feats_q, W_cls_ref[...], preferred_element_type=jnp.float32)
        + b_cls_ref[...])
    nq = jnp.sqrt(jnp.sum(feats_q * feats_q, axis=1, keepdims=True))
    q = feats_q / jnp.maximum(nq, 1e-12)
    q_ref[...] = q

    W_k2 = W_k_ref[...] * _M + W_q * (1.0 - _M)
    b_k2 = b_k_ref[...] * _M + b_q * (1.0 - _M)
    W_cls_k2 = W_cls_k_ref[...] * _M + W_cls_ref[...] * (1.0 - _M)
    b_cls_k2 = b_cls_k_ref[...] * _M + b_cls_ref[...] * (1.0 - _M)

    feats_k = jnp.dot(im_k_ref[...], W_k2, preferred_element_type=jnp.float32) + b_k2
    logits_k_u = (jnp.dot(feats_k, W_cls_k2, preferred_element_type=jnp.float32)
                  + b_cls_k2)
    logits_k = jnp.dot(P_ref[...], logits_k_u,
                       preferred_element_type=jnp.float32)
    logits_k_ref[...] = logits_k
    nk = jnp.sqrt(jnp.sum(feats_k * feats_k, axis=1, keepdims=True))
    k = feats_k / jnp.maximum(nk, 1e-12)
    k_ref[...] = k
    kT_ref[...] = k.T

    l_pos_ref[...] = jnp.sum(q * k, axis=1, keepdims=True) * (1.0 / _T)

    m = jnp.max(logits_k_u, axis=1, keepdims=True)
    e = jnp.exp(logits_k_u - m)
    p_u = e / jnp.sum(e, axis=1, keepdims=True)
    probs_ref[...] = jnp.dot(P_ref[...], p_u,
                             preferred_element_type=jnp.float32)
    col = jax.lax.broadcasted_iota(jnp.int32, (_B, _NCLS), 1)
    pseudo_ref[...] = jnp.min(jnp.where(logits_k_u == m, col, _NCLS),
                              axis=1, keepdims=True)


def _k2_body(memf_ref, q_ref, kT_ref, l_pos_ref, probs_ref,
             pseudo_ref, idxs_ref, memp_ref, meml_ref, memi_ref,
             lnn_ref, lins_ref, memf_out, memp_out, meml_out, memi_out,
             carry_ref):
    j = pl.program_id(0)

    @pl.when(j == 0)
    def _():
        carry_ref[...] = l_pos_ref[...]

    @pl.when(j < _NJ)
    def _():
        a = memf_ref[...]
        b = memf_ref[:, pl.ds(j * _TJ, _TJ)]
        lnn_ref[...] = jax.lax.dot_general(
            a, b, (((0,), (0,)), ((), ())), preferred_element_type=jnp.float32)
        lneg = (jnp.dot(q_ref[...], b, preferred_element_type=jnp.float32)
                * (1.0 / _T))
        lins_ref[...] = jnp.concatenate(
            [carry_ref[...], lneg[:, :_TJ - 1]], axis=1)
        carry_ref[...] = lneg[:, _TJ - 1:]

        memf_out[...] = b
        memp_out[...] = memp_ref[...]
        meml_out[...] = meml_ref[...]
        memi_out[...] = memi_ref[...]

        @pl.when(j == 0)
        def _():
            memf_out[:, 0:_B] = kT_ref[...]
            memp_out[0:_B, :] = probs_ref[...]
            meml_out[0, 0:2, :] = pseudo_ref[...]
            memi_out[0, 0:2, :] = idxs_ref[...]

    @pl.when(j == _NJ)
    def _():
        lins_ref[:, 0:1] = carry_ref[...]


def kernel(im_q, im_k, idxs, W_q, b_q, W_cls, b_cls, W_k, b_k, W_cls_k,
           b_cls_k, mem_feat, mem_labels, mem_probs, mem_index):
    f32 = jnp.float32

    P = jnp.asarray(_PERM_ONEHOT)
    (feats_q, logits_q, q, k, kT, logits_k, l_pos, probs,
     pseudo_u) = pl.pallas_call(
        _k1_body,
        out_shape=(
            jax.ShapeDtypeStruct((_B, _FEAT), f32),
            jax.ShapeDtypeStruct((_B, _NCLS), f32),
            jax.ShapeDtypeStruct((_B, _FEAT), f32),
            jax.ShapeDtypeStruct((_B, _FEAT), f32),
            jax.ShapeDtypeStruct((_FEAT, _B), f32),
            jax.ShapeDtypeStruct((_B, _NCLS), f32),
            jax.ShapeDtypeStruct((_B, 1), f32),
            jax.ShapeDtypeStruct((_B, _NCLS), f32),
            jax.ShapeDtypeStruct((_B, 1), jnp.int32),
        ),
    )(im_q, im_k, W_q, b_q.reshape(1, _FEAT), W_cls, b_cls.reshape(1, _NCLS),
      W_k, b_k.reshape(1, _FEAT), W_cls_k, b_cls_k.reshape(1, _NCLS), P)

    pseudo = pseudo_u[_PERM]

    meml2d = mem_labels.reshape(_NJ, _TJ // 128, 128)
    memi2d = mem_index.reshape(_NJ, _TJ // 128, 128)
    pseudo2d = pseudo.reshape(2, 128)
    idxs2d = idxs.astype(mem_index.dtype).reshape(2, 128)

    (l_neg_near, logits_ins, mem_feat_new, mem_probs_new, meml_new,
     memi_new) = pl.pallas_call(
        _k2_body,
        grid=(_NJ + 1,),
        scratch_shapes=[pltpu.VMEM((_B, 1), jnp.float32)],
        in_specs=[
            pl.BlockSpec((_FEAT, _K), lambda j: (0, 0)),
            pl.BlockSpec((_B, _FEAT), lambda j: (0, 0)),
            pl.BlockSpec((_FEAT, _B), lambda j: (0, 0)),
            pl.BlockSpec((_B, 1), lambda j: (0, 0)),
            pl.BlockSpec((_B, _NCLS), lambda j: (0, 0)),
            pl.BlockSpec((2, 128), lambda j: (0, 0)),
            pl.BlockSpec((2, 128), lambda j: (0, 0)),
            pl.BlockSpec((_TJ, _NCLS),
                         lambda j: (jnp.minimum(j, _NJ - 1), 0)),
            pl.BlockSpec((1, _TJ // 128, 128),
                         lambda j: (jnp.minimum(j, _NJ - 1), 0, 0)),
            pl.BlockSpec((1, _TJ // 128, 128),
                         lambda j: (jnp.minimum(j, _NJ - 1), 0, 0)),
        ],
        out_specs=[
            pl.BlockSpec((_K, _TJ), lambda j: (0, jnp.minimum(j, _NJ - 1))),
            pl.BlockSpec((_B, _TJ), lambda j: (0, j)),
            pl.BlockSpec((_FEAT, _TJ),
                         lambda j: (0, jnp.minimum(j, _NJ - 1))),
            pl.BlockSpec((_TJ, _NCLS),
                         lambda j: (jnp.minimum(j, _NJ - 1), 0)),
            pl.BlockSpec((1, _TJ // 128, 128),
                         lambda j: (jnp.minimum(j, _NJ - 1), 0, 0)),
            pl.BlockSpec((1, _TJ // 128, 128),
                         lambda j: (jnp.minimum(j, _NJ - 1), 0, 0)),
        ],
        out_shape=(
            jax.ShapeDtypeStruct((_K, _K), f32),
            jax.ShapeDtypeStruct((_B, _K + 1), f32),
            jax.ShapeDtypeStruct((_FEAT, _K), f32),
            jax.ShapeDtypeStruct((_K, _NCLS), f32),
            jax.ShapeDtypeStruct((_NJ, _TJ // 128, 128), mem_labels.dtype),
            jax.ShapeDtypeStruct((_NJ, _TJ // 128, 128), mem_index.dtype),
        ),
    )(mem_feat, q, kT, l_pos, probs, pseudo2d,
      idxs2d, mem_probs, meml2d, memi2d)

    return (feats_q, logits_q, logits_ins, k, logits_k, l_neg_near,
            mem_feat_new, meml_new.reshape(_K), mem_probs_new,
            memi_new.reshape(_K))


# speedup vs baseline: 3.2927x; 1.0141x over previous
import numpy as np
import jax
import jax.numpy as jnp
from jax.experimental import pallas as pl
from jax.experimental.pallas import tpu as pltpu

_K = 8192
_FEAT = 256
_NCLS = 65
_B = 256
_DIN = 2048
_M = 0.999
_T = 0.07

_TJ = 256
_NJ = _K // _TJ

_PERM = np.array([
    183, 138, 166, 19, 76, 158, 219, 118, 143, 54, 237, 189, 227, 149, 90, 30,
    7, 96, 139, 155, 131, 121, 115, 6, 35, 23, 58, 228, 128, 230, 16, 21,
    194, 213, 156, 220, 77, 154, 160, 94, 116, 61, 229, 38, 3, 185, 105, 132,
    81, 26, 32, 64, 37, 56, 51, 2, 193, 122, 248, 63, 133, 52, 20, 89,
    202, 95, 44, 47, 123, 239, 79, 84, 222, 144, 157, 135, 50, 242, 140, 78,
    179, 72, 163, 191, 83, 42, 62, 254, 152, 69, 235, 53, 247, 234, 245, 223,
    148, 172, 215, 0, 201, 226, 145, 8, 208, 203, 167, 169, 159, 251, 109, 181,
    22, 178, 13, 29, 99, 110, 244, 34, 70, 175, 18, 103, 196, 141, 252, 232,
    86, 142, 75, 233, 198, 187, 206, 91, 243, 111, 24, 113, 1, 65, 48, 5,
    238, 45, 199, 165, 150, 49, 173, 214, 236, 33, 216, 74, 55, 182, 136, 60,
    204, 119, 57, 124, 27, 112, 129, 249, 209, 151, 10, 134, 192, 246, 186, 93,
    176, 161, 68, 146, 240, 15, 217, 73, 241, 40, 210, 67, 88, 102, 107, 66,
    80, 100, 120, 211, 147, 71, 207, 17, 59, 184, 98, 225, 108, 114, 36, 125,
    101, 218, 180, 92, 171, 153, 28, 46, 9, 104, 200, 255, 117, 221, 4, 177,
    170, 190, 130, 12, 168, 195, 188, 87, 85, 212, 14, 174, 82, 31, 106, 127,
    250, 162, 126, 164, 231, 97, 224, 41, 253, 137, 197, 25, 43, 39, 11, 205,
], dtype=np.int32)

_PERM_ONEHOT = np.zeros((_B, _B), dtype=np.float32)
_PERM_ONEHOT[np.arange(_B), _PERM] = 1.0


def _body(im_q_ref, im_k_ref, W_q_ref, b_q_ref, W_cls_ref, b_cls_ref,
          W_k_ref, b_k_ref, W_cls_k_ref, b_cls_k_ref, P_ref, memf_ref,
          idxs_ref, memp_ref, meml_ref, memi_ref,
          fq_ref, lq_ref, k_ref, lk_ref, lnn_ref, lins_ref,
          memf_out, memp_out, meml_out, memi_out,
          q_s, kT_s, probs_s, pseudo_s, carry_s):
    j = pl.program_id(0)

    @pl.when(j == 0)
    def _():
        im_q = im_q_ref[...]
        W_q = W_q_ref[...]
        b_q = b_q_ref[...]
        feats_q = jnp.dot(im_q, W_q, preferred_element_type=jnp.float32) + b_q
        fq_ref[...] = feats_q
        lq_ref[...] = (jnp.dot(feats_q, W_cls_ref[...],
                               preferred_element_type=jnp.float32)
                       + b_cls_ref[...])
        nq = jnp.sqrt(jnp.sum(feats_q * feats_q, axis=1, keepdims=True))
        q = feats_q / jnp.maximum(nq, 1e-12)
        q_s[...] = q

        W_k2 = W_k_ref[...] * _M + W_q * (1.0 - _M)
        b_k2 = b_k_ref[...] * _M + b_q * (1.0 - _M)
        W_cls_k2 = W_cls_k_ref[...] * _M + W_cls_ref[...] * (1.0 - _M)
        b_cls_k2 = b_cls_k_ref[...] * _M + b_cls_ref[...] * (1.0 - _M)

        P = P_ref[...]
        feats_k = (jnp.dot(im_k_ref[...], W_k2,
                           preferred_element_type=jnp.float32) + b_k2)
        lk_u = (jnp.dot(feats_k, W_cls_k2, preferred_element_type=jnp.float32)
                + b_cls_k2)
        lk_ref[...] = jnp.dot(P, lk_u, preferred_element_type=jnp.float32)
        nk = jnp.sqrt(jnp.sum(feats_k * feats_k, axis=1, keepdims=True))
        k = feats_k / jnp.maximum(nk, 1e-12)
        k_ref[...] = k
        kT_s[...] = k.T

        carry_s[...] = jnp.sum(q * k, axis=1, keepdims=True) * (1.0 / _T)

        m = jnp.max(lk_u, axis=1, keepdims=True)
        e = jnp.exp(lk_u - m)
        p_u = e / jnp.sum(e, axis=1, keepdims=True)
        probs_s[...] = jnp.dot(P, p_u, preferred_element_type=jnp.float32)
        col = jax.lax.broadcasted_iota(jnp.int32, (_B, _NCLS), 1)
        pseudo_u = jnp.min(jnp.where(lk_u == m, col, _NCLS),
                           axis=1, keepdims=True)
        pr = jnp.dot(P, pseudo_u.astype(jnp.float32),
                     preferred_element_type=jnp.float32)
        pseudo_s[...] = (pr + 0.5).astype(jnp.int32).reshape(2, 128)

    @pl.when(j < _NJ)
    def _():
        a = memf_ref[...]
        b = memf_ref[:, pl.ds(j * _TJ, _TJ)]
        lnn_ref[...] = jax.lax.dot_general(
            a, b, (((0,), (0,)), ((), ())), preferred_element_type=jnp.float32)
        lneg = (jnp.dot(q_s[...], b, preferred_element_type=jnp.float32)
                * (1.0 / _T))
        lins_ref[...] = jnp.concatenate(
            [carry_s[...], lneg[:, :_TJ - 1]], axis=1)
        carry_s[...] = lneg[:, _TJ - 1:]

        memf_out[...] = b
        memp_out[...] = memp_ref[...]
        meml_out[...] = meml_ref[...]
        memi_out[...] = memi_ref[...]

        @pl.when(j == 0)
        def _():
            memf_out[...] = kT_s[...]
            memp_out[...] = probs_s[...]
            meml_out[0] = pseudo_s[...]
            memi_out[0] = idxs_ref[...]

    @pl.when(j == _NJ)
    def _():
        lins_ref[:, 0:1] = carry_s[...]


def kernel(im_q, im_k, idxs, W_q, b_q, W_cls, b_cls, W_k, b_k, W_cls_k,
           b_cls_k, mem_feat, mem_labels, mem_probs, mem_index):
    f32 = jnp.float32
    P = jnp.asarray(_PERM_ONEHOT)
    meml3d = mem_labels.reshape(_NJ, _TJ // 128, 128)
    memi3d = mem_index.reshape(_NJ, _TJ // 128, 128)
    idxs2d = idxs.astype(mem_index.dtype).reshape(2, 128)
    _c = lambda j: (0, 0)
    _cj = lambda j: (jnp.minimum(j, _NJ - 1), 0)
    _cj3 = lambda j: (jnp.minimum(j, _NJ - 1), 0, 0)

    (feats_q, logits_q, k, logits_k, l_neg_near, logits_ins, mem_feat_new,
     mem_probs_new, meml_new, memi_new) = pl.pallas_call(
        _body,
        grid=(_NJ + 1,),
        in_specs=[
            pl.BlockSpec((_B, _DIN), _c),
            pl.BlockSpec((_B, _DIN), _c),
            pl.BlockSpec((_DIN, _FEAT), _c),
            pl.BlockSpec((1, _FEAT), _c),
            pl.BlockSpec((_FEAT, _NCLS), _c),
            pl.BlockSpec((1, _NCLS), _c),
            pl.BlockSpec((_DIN, _FEAT), _c),
            pl.BlockSpec((1, _FEAT), _c),
            pl.BlockSpec((_FEAT, _NCLS), _c),
            pl.BlockSpec((1, _NCLS), _c),
            pl.BlockSpec((_B, _B), _c),
            pl.BlockSpec((_FEAT, _K), _c),
            pl.BlockSpec((2, 128), _c),
            pl.BlockSpec((_TJ, _NCLS), _cj),
            pl.BlockSpec((1, _TJ // 128, 128), _cj3),
            pl.BlockSpec((1, _TJ // 128, 128), _cj3),
        ],
        out_specs=[
            pl.BlockSpec((_B, _FEAT), _c),
            pl.BlockSpec((_B, _NCLS), _c),
            pl.BlockSpec((_B, _FEAT), _c),
            pl.BlockSpec((_B, _NCLS), _c),
            pl.BlockSpec((_K, _TJ),
                         lambda j: (0, jnp.minimum(j, _NJ - 1))),
            pl.BlockSpec((_B, _TJ), lambda j: (0, j)),
            pl.BlockSpec((_FEAT, _TJ),
                         lambda j: (0, jnp.minimum(j, _NJ - 1))),
            pl.BlockSpec((_TJ, _NCLS), _cj),
            pl.BlockSpec((1, _TJ // 128, 128), _cj3),
            pl.BlockSpec((1, _TJ // 128, 128), _cj3),
        ],
        out_shape=(
            jax.ShapeDtypeStruct((_B, _FEAT), f32),
            jax.ShapeDtypeStruct((_B, _NCLS), f32),
            jax.ShapeDtypeStruct((_B, _FEAT), f32),
            jax.ShapeDtypeStruct((_B, _NCLS), f32),
            jax.ShapeDtypeStruct((_K, _K), f32),
            jax.ShapeDtypeStruct((_B, _K + 1), f32),
            jax.ShapeDtypeStruct((_FEAT, _K), f32),
            jax.ShapeDtypeStruct((_K, _NCLS), f32),
            jax.ShapeDtypeStruct((_NJ, _TJ // 128, 128), mem_labels.dtype),
            jax.ShapeDtypeStruct((_NJ, _TJ // 128, 128), mem_index.dtype),
        ),
        scratch_shapes=[
            pltpu.VMEM((_B, _FEAT), f32),
            pltpu.VMEM((_FEAT, _B), f32),
            pltpu.VMEM((_B, _NCLS), f32),
            pltpu.VMEM((2, 128), jnp.int32),
            pltpu.VMEM((_B, 1), f32),
        ],
    )(im_q, im_k, W_q, b_q.reshape(1, _FEAT), W_cls, b_cls.reshape(1, _NCLS),
      W_k, b_k.reshape(1, _FEAT), W_cls_k, b_cls_k.reshape(1, _NCLS), P,
      mem_feat, idxs2d, mem_probs, meml3d, memi3d)

    return (feats_q, logits_q, logits_ins, k, logits_k, l_neg_near,
            mem_feat_new, meml_new.reshape(_K), mem_probs_new,
            memi_new.reshape(_K))
